# Initial kernel scaffold; baseline (speedup 1.0000x reference)
import jax
import jax.numpy as jnp
from jax.experimental import pallas as pl


def kernel(x, edge_index, W1, b1, gamma, beta, W2, b2):
    raise NotImplementedError("write your pallas kernel here")


# speedup vs baseline: 5.6032x; 5.6032x over previous
import functools

import jax
import jax.numpy as jnp
from jax import lax
from jax.experimental import pallas as pl
from jax.experimental.pallas import tpu as pltpu
from jax.experimental.pallas import tpu_sc as plsc

N = 10000
D = 256
H = 128
E = 160000
K = 10
ALPHA = 0.1

B = 128
EP = 163840
NCHUNK = EP // B
ACC_ROWS = 10112
ZROWS = 632
RB = 1000

_HIGH = jax.lax.Precision.HIGHEST


def _zero_fill(buf, nrows, ncols):
    def body(i, _):
        for j in range(ncols // 16):
            buf[i, pl.ds(j * 16, 16)] = jnp.zeros((16,), jnp.float32)
        return 0
    lax.fori_loop(0, nrows, body, 0, unroll=False)


def _deg_body(dstI, degp, dacc, ones_v, idx_v, zb):
    cid = lax.axis_index("c")
    sid = lax.axis_index("s")

    def core(c):
        _zero_fill(zb, 128, 16)
        def ones_body(i, _):
            ones_v[i, pl.ds(0, 16)] = jnp.ones((16,), jnp.float32)
            return 0
        lax.fori_loop(0, 128, ones_body, 0, unroll=False)
        base = sid * ZROWS
        for off, sz in ((0, 128), (128, 128), (256, 128), (384, 128), (512, 120)):
            pltpu.sync_copy(zb.at[pl.ds(0, sz)], dacc.at[pl.ds(base + off, sz)])
        plsc.subcore_barrier()
        def chunk(r, _):
            pltpu.sync_copy(dstI.at[r], idx_v)
            pltpu.sync_copy(ones_v, dacc.at[idx_v], add=True)
            return 0
        r0 = c * (NCHUNK // 2) + sid * (NCHUNK // 32)
        lax.fori_loop(r0, r0 + NCHUNK // 32, chunk, 0, unroll=False)
        plsc.subcore_barrier()
        pltpu.sync_copy(dacc.at[pl.ds(base, ZROWS)],
                        degp.at[c].at[pl.ds(base, ZROWS)])

    @pl.when(cid == 0)
    def _():
        core(0)

    @pl.when(cid == 1)
    def _():
        core(1)


_deg_kernel = pl.kernel(
    _deg_body,
    out_type=jax.ShapeDtypeStruct((2, ACC_ROWS, 16), jnp.float32),
    mesh=plsc.VectorSubcoreMesh(core_axis_name="c", subcore_axis_name="s", num_cores=2, num_subcores=16),
    scratch_types=[
        pltpu.VMEM_SHARED((ACC_ROWS, 16), jnp.float32),
        pltpu.VMEM((128, 16), jnp.float32),
        pltpu.VMEM((B,), jnp.int32),
        pltpu.VMEM((128, 16), jnp.float32),
    ],
)


NBUF = 2
TCHUNK = NCHUNK // 16
HCHUNK = TCHUNK // 2
TSTEP = HCHUNK // NBUF
ZBUF = 40


def _edge_body(g0, g1, srcI, dstI, s2, acc, zb, rows, srcb, dstb, gsems, ssems):
    cid = lax.axis_index("c")
    sid = lax.axis_index("s")

    def core(gref, c):
        _zero_fill(zb, ZBUF, H)
        base = sid * ZROWS
        for zi in range(15):
            pltpu.sync_copy(zb, acc.at[pl.ds(base + zi * ZBUF, ZBUF)])
        pltpu.sync_copy(zb.at[pl.ds(0, 32)], acc.at[pl.ds(base + 600, 32)])
        plsc.subcore_barrier()

        for half in range(2):
            r0 = sid * TCHUNK + half * HCHUNK
            pltpu.sync_copy(srcI.at[pl.ds(r0, HCHUNK)], srcb)
            pltpu.sync_copy(dstI.at[pl.ds(r0, HCHUNK)], dstb)

            def step(i, _):
                j = i * NBUF
                gds = []
                for b in range(NBUF):
                    gds.append(pltpu.async_copy(
                        gref.at[srcb.at[j + b]], rows.at[b], gsems.at[b]))
                sds = []
                for b in range(NBUF):
                    gds[b].wait()
                    sds.append(pltpu.async_copy(
                        rows.at[b], acc.at[dstb.at[j + b]], ssems.at[b],
                        add=True))
                for b in range(NBUF):
                    sds[b].wait()
                return 0

            lax.fori_loop(0, TSTEP, step, 0, unroll=False)
        plsc.subcore_barrier()
        pltpu.sync_copy(acc.at[pl.ds(base, ZROWS)],
                        s2.at[c].at[pl.ds(base, ZROWS)])

    @pl.when(cid == 0)
    def _():
        core(g0, 0)

    @pl.when(cid == 1)
    def _():
        core(g1, 1)


_edge_kernel = pl.kernel(
    _edge_body,
    out_type=jax.ShapeDtypeStruct((2, ACC_ROWS, H), jnp.float32),
    mesh=plsc.VectorSubcoreMesh(core_axis_name="c", subcore_axis_name="s", num_cores=2, num_subcores=16),
    scratch_types=[
        pltpu.VMEM_SHARED((ACC_ROWS, H), jnp.float32),
        pltpu.VMEM((ZBUF, H), jnp.float32),
        pltpu.VMEM((NBUF, B, H), jnp.float32),
        pltpu.VMEM((HCHUNK, B), jnp.int32),
        pltpu.VMEM((HCHUNK, B), jnp.int32),
        pltpu.SemaphoreType.DMA((NBUF,)),
        pltpu.SemaphoreType.DMA((NBUF,)),
    ],
)


def _mlp1_body(x_ref, w1_ref, b1_ref, z_ref, st_ref):
    i = pl.program_id(0)
    z = jnp.dot(x_ref[...], w1_ref[...], preferred_element_type=jnp.float32,
                precision=_HIGH) + b1_ref[...]
    z_ref[...] = z
    s1 = jnp.sum(z, axis=0, keepdims=True)
    s2 = jnp.sum(z * z, axis=0, keepdims=True)
    contrib = jnp.concatenate([s1, s2, jnp.zeros((6, D), jnp.float32)], axis=0)

    @pl.when(i == 0)
    def _():
        st_ref[...] = jnp.zeros((8, D), jnp.float32)

    st_ref[...] += contrib


def _prep_body(degp_ref, c1_ref, dinv_ref, sdeg_ref):
    dp = degp_ref[...]
    deg = dp[0, :, 0:1] + dp[1, :, 0:1] + 1.0
    degb = jnp.broadcast_to(deg, (RB, H))
    c1_ref[...] = (1.0 - ALPHA) / degb
    dinv_ref[...] = lax.rsqrt(degb)
    sdeg_ref[...] = jnp.sqrt(degb)


def _mlp2_body(z_ref, st_ref, gamma_ref, beta_ref, w2_ref, b2_ref, dinv_ref,
               g0_ref, g1_ref, a00_ref, a01_ref):
    st = st_ref[...]
    mean = st[0:1, :] / N
    var = st[1:2, :] / N - mean * mean
    rstd = lax.rsqrt(var + 1e-5)
    zn = (z_ref[...] - mean) * rstd * gamma_ref[...] + beta_ref[...]
    hb = jnp.maximum(zn, 0.0)
    h = jnp.dot(hb, w2_ref[...], preferred_element_type=jnp.float32,
                precision=_HIGH) + b2_ref[...]
    dv = dinv_ref[...]
    g0 = h[:, :H] * dv
    g1 = h[:, H:] * dv
    g0_ref[...] = g0
    g1_ref[...] = g1
    a00_ref[...] = ALPHA * g0
    a01_ref[...] = ALPHA * g1


def _upd_body(s2_ref, g0_ref, g1_ref, a00_ref, a01_ref, c1_ref,
              ng0_ref, ng1_ref):
    c1 = c1_ref[...]
    ng0_ref[...] = c1 * (s2_ref[0] + g0_ref[...]) + a00_ref[...]
    ng1_ref[...] = c1 * (s2_ref[1] + g1_ref[...]) + a01_ref[...]


def _final_body(x_ref, s2_ref, g0_ref, g1_ref, a00_ref, a01_ref, c1_ref,
                sdeg_ref, out_ref):
    c1 = c1_ref[...]
    sd = sdeg_ref[...]
    h0f = (c1 * (s2_ref[0] + g0_ref[...]) + a00_ref[...]) * sd
    h1f = (c1 * (s2_ref[1] + g1_ref[...]) + a01_ref[...]) * sd
    hK = jnp.concatenate([h0f, h1f], axis=1)
    out_ref[...] = x_ref[...] + jnp.maximum(hK, 0.0)


def _row_spec(cols):
    return pl.BlockSpec((RB, cols), lambda i: (i, 0))


def _full_spec(shape):
    return pl.BlockSpec(shape, lambda i: tuple(0 for _ in shape))


_GRID = N // RB

_mlp1 = pl.pallas_call(
    _mlp1_body,
    grid=(_GRID,),
    in_specs=[_row_spec(D), _full_spec((D, D)), _full_spec((1, D))],
    out_specs=[_row_spec(D), _full_spec((8, D))],
    out_shape=[jax.ShapeDtypeStruct((N, D), jnp.float32),
               jax.ShapeDtypeStruct((8, D), jnp.float32)],
)

_prep = pl.pallas_call(
    _prep_body,
    grid=(_GRID,),
    in_specs=[pl.BlockSpec((2, RB, 16), lambda i: (0, i, 0))],
    out_specs=[_row_spec(H), _row_spec(H), _row_spec(H)],
    out_shape=[jax.ShapeDtypeStruct((N, H), jnp.float32)] * 3,
)

_mlp2 = pl.pallas_call(
    _mlp2_body,
    grid=(_GRID,),
    in_specs=[_row_spec(D), _full_spec((8, D)), _full_spec((1, D)),
              _full_spec((1, D)), _full_spec((D, D)), _full_spec((1, D)),
              _row_spec(H)],
    out_specs=[_row_spec(H)] * 4,
    out_shape=[jax.ShapeDtypeStruct((N, H), jnp.float32)] * 4,
)

_upd = pl.pallas_call(
    _upd_body,
    grid=(_GRID,),
    in_specs=[pl.BlockSpec((2, RB, H), lambda i: (0, i, 0))] + [_row_spec(H)] * 5,
    out_specs=[_row_spec(H)] * 2,
    out_shape=[jax.ShapeDtypeStruct((N, H), jnp.float32)] * 2,
)

_final = pl.pallas_call(
    _final_body,
    grid=(_GRID,),
    in_specs=[_row_spec(D), pl.BlockSpec((2, RB, H), lambda i: (0, i, 0))]
             + [_row_spec(H)] * 6,
    out_specs=_row_spec(D),
    out_shape=jax.ShapeDtypeStruct((N, D), jnp.float32),
)


def kernel(x, edge_index, W1, b1, gamma, beta, W2, b2):
    src = edge_index[0].astype(jnp.int32)
    dst = edge_index[1].astype(jnp.int32)
    pad = EP - E
    srcP = jnp.concatenate([src, jnp.zeros((pad,), jnp.int32)]).reshape(NCHUNK, B)
    dstP = jnp.concatenate([dst, jnp.full((pad,), N, jnp.int32)]).reshape(NCHUNK, B)
    b1r = b1.reshape(1, D)
    b2r = b2.reshape(1, D)
    gammar = gamma.reshape(1, D)
    betar = beta.reshape(1, D)

    degp = _deg_kernel(dstP)
    z, st = _mlp1(x, W1, b1r)
    c1, dinv, sdeg = _prep(degp)
    g0, g1, a00, a01 = _mlp2(z, st, gammar, betar, W2, b2r, dinv)

    for _ in range(K - 1):
        s2 = _edge_kernel(g0, g1, srcP, dstP)
        g0, g1 = _upd(s2, g0, g1, a00, a01, c1)
    s2 = _edge_kernel(g0, g1, srcP, dstP)
    out = _final(x, s2, g0, g1, a00, a01, c1, sdeg)
    return out
